# baseline (speedup 1.0000x reference)
import functools

import jax
import jax.numpy as jnp
from jax import lax
from jax.experimental import pallas as pl
from jax.experimental.pallas import tpu as pltpu
from jax.experimental.pallas import tpu_sc as plsc

VOCAB = 1000
D = 1000
B = 1024
T = 50
TP = 56
NC = 2
NS = 16
NW = NC * NS
NVT = D // 8


def _make_gather():
    mesh = plsc.VectorSubcoreMesh(core_axis_name="c", subcore_axis_name="s")

    @functools.partial(
        pl.kernel,
        mesh=mesh,
        compiler_params=pltpu.CompilerParams(needs_layout_passes=False),
        out_type=jax.ShapeDtypeStruct((T, D, B), jnp.float32),
        scratch_types=[
            pltpu.VMEM((TP, B), jnp.int32),
            pltpu.VMEM((8, B), jnp.float32),
            pltpu.VMEM((1, 8, B), jnp.float32),
            pltpu.VMEM((1, 8, B), jnp.float32),
            pltpu.SemaphoreType.DMA,
            pltpu.SemaphoreType.DMA,
        ],
    )
    def k(idx_hbm, tableT_hbm, out_hbm, idx_v, slab, os0, os1, o0, o1):
        wid = lax.axis_index("s") * NC + lax.axis_index("c")
        lo = wid * NVT // NW
        hi = (wid + 1) * NVT // NW
        pltpu.sync_copy(idx_hbm, idx_v)

        outslabs = (os0, os1)
        osems = (o0, o1)
        rvecs = [jnp.full((16,), r, dtype=jnp.int32) for r in range(8)]

        @pl.loop(lo, hi)
        def _(vt):
            v8 = pl.multiple_of(vt * 8, 8)
            pltpu.sync_copy(tableT_hbm.at[pl.ds(v8, 8)], slab)

            @pl.loop(0, T, step=2)
            def _(c):
                for b in range(2):
                    t = c + b
                    outslab = outslabs[b]
                    dst = out_hbm.at[pl.ds(t, 1), pl.ds(v8, 8), :]
                    @pl.when(t >= 2)
                    def _():
                        pltpu.make_async_copy(outslab, dst, osems[b]).wait()
                    for kk in range(B // 16):
                        idx16 = idx_v[t, pl.ds(16 * kk, 16)]
                        for r in range(8):
                            outslab[0, r, pl.ds(16 * kk, 16)] = (
                                plsc.load_gather(slab, [rvecs[r], idx16]))
                    pltpu.async_copy(outslab, dst, osems[b])

            for b in range(2):
                pltpu.make_async_copy(
                    outslabs[b],
                    out_hbm.at[pl.ds(T - 2 + b, 1), pl.ds(v8, 8), :],
                    osems[b]).wait()

    return k


_gather = _make_gather()


def kernel(index, table):
    idx_t = jnp.pad(index.T.astype(jnp.int32), ((0, TP - T), (0, 0)))
    table_tp = jnp.pad(table.T, ((0, 0), (0, B - VOCAB)))
    out2 = _gather(idx_t, table_tp)
    return jnp.transpose(out2, (2, 0, 1))


# speedup vs baseline: 5.6884x; 5.6884x over previous
import functools

import jax
import jax.numpy as jnp
from jax import lax
from jax.experimental import pallas as pl
from jax.experimental.pallas import tpu as pltpu
from jax.experimental.pallas import tpu_sc as plsc

VOCAB = 1000
D = 1000
B = 1024
T = 50
TP = 56
NC = 2
NS = 16
NW = NC * NS
NVT = D // 8


def _make_gather():
    mesh = plsc.VectorSubcoreMesh(core_axis_name="c", subcore_axis_name="s")

    @functools.partial(
        pl.kernel,
        mesh=mesh,
        compiler_params=pltpu.CompilerParams(needs_layout_passes=False),
        out_type=jax.ShapeDtypeStruct((T, D, B), jnp.float32),
        scratch_types=[
            pltpu.VMEM((TP, B), jnp.int32),
            pltpu.VMEM((8, B), jnp.float32),
            pltpu.VMEM((1, 8, B), jnp.float32),
            pltpu.VMEM((1, 8, B), jnp.float32),
            pltpu.SemaphoreType.DMA,
            pltpu.SemaphoreType.DMA,
        ],
    )
    def k(idx_hbm, tableT_hbm, out_hbm, idx_v, slab, os0, os1, o0, o1):
        wid = lax.axis_index("s") * NC + lax.axis_index("c")
        lo = wid * NVT // NW
        hi = (wid + 1) * NVT // NW
        pltpu.sync_copy(idx_hbm, idx_v)

        outslabs = (os0, os1)
        osems = (o0, o1)
        rvecs = [jnp.full((16,), r, dtype=jnp.int32) for r in range(8)]

        @pl.loop(lo, hi)
        def _(vt):
            v8 = pl.multiple_of(vt * 8, 8)
            pltpu.sync_copy(tableT_hbm.at[pl.ds(v8, 8)], slab)

            @pl.loop(0, T, step=2)
            def _(c):
                for b in range(2):
                    t = c + b
                    outslab = outslabs[b]
                    dst = out_hbm.at[pl.ds(t, 1), pl.ds(v8, 8), :]
                    @pl.when(t >= 2)
                    def _():
                        pltpu.make_async_copy(outslab, dst, osems[b]).wait()
                    @plsc.parallel_loop(0, B, step=16, unroll=4)
                    def _(col):
                        idx16 = idx_v[t, pl.ds(col, 16)]
                        for r in range(8):
                            outslab[0, r, pl.ds(col, 16)] = (
                                plsc.load_gather(slab, [rvecs[r], idx16]))
                    pltpu.async_copy(outslab, dst, osems[b])

            for b in range(2):
                pltpu.make_async_copy(
                    outslabs[b],
                    out_hbm.at[pl.ds(T - 2 + b, 1), pl.ds(v8, 8), :],
                    osems[b]).wait()

    return k


_gather = _make_gather()


def kernel(index, table):
    idx_t = jnp.pad(index.T.astype(jnp.int32), ((0, TP - T), (0, 0)))
    table_tp = jnp.pad(table.T, ((0, 0), (0, B - VOCAB)))
    out2 = _gather(idx_t, table_tp)
    return jnp.transpose(out2, (2, 0, 1))


# speedup vs baseline: 5.7479x; 1.0105x over previous
import functools

import jax
import jax.numpy as jnp
from jax import lax
from jax.experimental import pallas as pl
from jax.experimental.pallas import tpu as pltpu
from jax.experimental.pallas import tpu_sc as plsc

VOCAB = 1000
D = 1000
B = 1024
T = 50
TP = 56
NC = 2
NS = 16
NW = NC * NS
NVT = D // 8


def _make_gather():
    mesh = plsc.VectorSubcoreMesh(core_axis_name="c", subcore_axis_name="s")

    @functools.partial(
        pl.kernel,
        mesh=mesh,
        compiler_params=pltpu.CompilerParams(needs_layout_passes=False),
        out_type=jax.ShapeDtypeStruct((T, D, B), jnp.float32),
        scratch_types=[
            pltpu.VMEM((TP, B), jnp.int32),
            pltpu.VMEM((8, B), jnp.float32),
            pltpu.VMEM((1, 8, B), jnp.float32),
            pltpu.VMEM((1, 8, B), jnp.float32),
            pltpu.SemaphoreType.DMA,
            pltpu.SemaphoreType.DMA,
        ],
    )
    def k(idx_hbm, tableT_hbm, out_hbm, idx_v, slab, os0, os1, o0, o1):
        wid = lax.axis_index("s") * NC + lax.axis_index("c")
        lo = wid * NVT // NW
        hi = (wid + 1) * NVT // NW
        pltpu.sync_copy(idx_hbm, idx_v)

        outslabs = (os0, os1)
        osems = (o0, o1)
        rvecs = [jnp.full((16,), r, dtype=jnp.int32) for r in range(8)]

        @pl.loop(lo, hi)
        def _(vt):
            v8 = pl.multiple_of(vt * 8, 8)
            pltpu.sync_copy(tableT_hbm.at[pl.ds(v8, 8)], slab)

            @pl.loop(0, T, step=2)
            def _(c):
                for b in range(2):
                    t = c + b
                    outslab = outslabs[b]
                    dst = out_hbm.at[pl.ds(t, 1), pl.ds(v8, 8), :]
                    @pl.when(t >= 2)
                    def _():
                        pltpu.make_async_copy(outslab, dst, osems[b]).wait()
                    @plsc.parallel_loop(0, B, step=16, unroll=8)
                    def _(col):
                        idx16 = idx_v[t, pl.ds(col, 16)]
                        for r in range(8):
                            outslab[0, r, pl.ds(col, 16)] = (
                                plsc.load_gather(slab, [rvecs[r], idx16]))
                    pltpu.async_copy(outslab, dst, osems[b])

            for b in range(2):
                pltpu.make_async_copy(
                    outslabs[b],
                    out_hbm.at[pl.ds(T - 2 + b, 1), pl.ds(v8, 8), :],
                    osems[b]).wait()

    return k


_gather = _make_gather()


def kernel(index, table):
    idx_t = jnp.pad(index.T.astype(jnp.int32), ((0, TP - T), (0, 0)))
    table_tp = jnp.pad(table.T, ((0, 0), (0, B - VOCAB)))
    out2 = _gather(idx_t, table_tp)
    return jnp.transpose(out2, (2, 0, 1))
